# baseline (speedup 1.0000x reference)
import functools

import jax
import jax.numpy as jnp
from jax import lax
from jax.experimental import pallas as pl
from jax.experimental.pallas import tpu as pltpu
from jax.experimental.pallas import tpu_sc as plsc

B, S, E = 4, 8192, 64
L = 16
NC, NS = 2, 16
NW = NC * NS
CHUNK = (B * S) // NW
W_PER_ROW = NW // B
NBLK = CHUNK // L
GCH = 128
NG = CHUNK // GCH
K = 4096

_mesh = plsc.VectorSubcoreMesh(core_axis_name="c", subcore_axis_name="s")


@functools.partial(
    pl.kernel,
    out_type=jax.ShapeDtypeStruct((B * S, E), jnp.float32),
    mesh=_mesh,
    scratch_types=[
        pltpu.VMEM((CHUNK,), jnp.int32),
        pltpu.VMEM((NG, GCH), jnp.int32),
        pltpu.VMEM((CHUNK, E), jnp.float32),
        pltpu.VMEM((L,), jnp.int32),
        pltpu.VMEM((NS, L), jnp.int32),
        pltpu.VMEM_SHARED((NS, L), jnp.int32),
        pltpu.SemaphoreType.DMA,
        pltpu.SemaphoreType.DMA,
    ],
    compiler_params=pltpu.CompilerParams(
        needs_layout_passes=False, use_tc_tiling_on_sc=False
    ),
)
def _pos_embed(mask_hbm, table_hbm, out_hbm, mchunk, pos, rows,
               tstage, totals, shared_totals, gsem, ssem):
    sid = lax.axis_index("s")
    wid = lax.axis_index("c") * NS + sid
    chunkid = sid % W_PER_ROW
    rowhalf = sid // W_PER_ROW

    pltpu.sync_copy(mask_hbm.at[pl.ds(wid * CHUNK, CHUNK)], mchunk)

    def _acc(i, acc):
        return acc + mchunk[pl.ds(i * L, L)]

    acc = lax.fori_loop(0, NBLK, _acc, jnp.zeros((L,), jnp.int32))
    total = jnp.sum(acc)

    tstage[...] = jnp.broadcast_to(total, (L,))
    pltpu.sync_copy(tstage, shared_totals.at[sid])
    plsc.subcore_barrier()
    pltpu.sync_copy(shared_totals, totals)

    iota_v = lax.iota(jnp.int32, L)
    row_v = rowhalf * W_PER_ROW + jnp.minimum(iota_v, W_PER_ROW - 1)
    tot_v = plsc.load_gather(totals, [row_v, jnp.zeros((L,), jnp.int32)])
    base0 = jnp.sum(jnp.where(iota_v < chunkid, tot_v, 0))

    def _blocks(j, base):
        for k in range(GCH // L):
            i = j * (GCH // L) + k
            v = mchunk[pl.ds(i * L, L)]
            cs = plsc.cumsum(v) + base
            q = jnp.minimum(cs, S - 1)
            alt = S + ((lax.iota(jnp.int32, L) + i * L + wid * 128) & (K - 1))
            pos[j, pl.ds(k * L, L)] = jnp.where(v > 0, q, alt)
            base = cs[L - 1]
        return base

    def _gather(j):
        return pltpu.async_copy(
            table_hbm.at[pos.at[j]], rows.at[pl.ds(j * GCH, GCH)], gsem
        )

    def _scatter(j):
        return pltpu.async_copy(
            rows.at[pl.ds(j * GCH, GCH)],
            out_hbm.at[pl.ds(wid * CHUNK + j * GCH, GCH)],
            ssem,
        )

    half = NG // 2
    base = base0
    g = []
    for j in range(half):
        base = _blocks(j, base)
        g.append(_gather(j))
    for j in range(half, NG):
        base = _blocks(j, base)
        g.append(_gather(j))
    sc = []
    for j in range(half):
        g[j].wait()
    for j in range(half):
        sc.append(_scatter(j))
    for j in range(half, NG):
        g[j].wait()
    for j in range(half, NG):
        sc.append(_scatter(j))
    for cp in sc:
        cp.wait()


def kernel(input, mask, table):
    del input
    table_aug = jnp.concatenate(
        [table, jnp.broadcast_to(table[0:1], (K, E))], axis=0
    )
    out = _pos_embed(mask.reshape(-1), table_aug)
    return out.reshape(B, S, E)


# speedup vs baseline: 1.0507x; 1.0507x over previous
import functools

import jax
import jax.numpy as jnp
from jax import lax
from jax.experimental import pallas as pl
from jax.experimental.pallas import tpu as pltpu
from jax.experimental.pallas import tpu_sc as plsc

B, S, E = 4, 8192, 64
L = 16
NC, NS = 2, 16
NW = NC * NS
CHUNK = (B * S) // NW
W_PER_ROW = NW // B
NBLK = CHUNK // L
GCH = 128
NG = CHUNK // GCH
K = 2048

_mesh = plsc.VectorSubcoreMesh(core_axis_name="c", subcore_axis_name="s")


@functools.partial(
    pl.kernel,
    out_type=jax.ShapeDtypeStruct((B * S, E), jnp.float32),
    mesh=_mesh,
    scratch_types=[
        pltpu.VMEM((CHUNK,), jnp.int32),
        pltpu.VMEM((NG, GCH), jnp.int32),
        pltpu.VMEM((CHUNK, E), jnp.float32),
        pltpu.VMEM((L,), jnp.int32),
        pltpu.VMEM((NS, L), jnp.int32),
        pltpu.VMEM_SHARED((NS, L), jnp.int32),
        pltpu.SemaphoreType.DMA,
        pltpu.SemaphoreType.DMA,
    ],
    compiler_params=pltpu.CompilerParams(
        needs_layout_passes=False, use_tc_tiling_on_sc=False
    ),
)
def _pos_embed(mask_hbm, table_hbm, out_hbm, mchunk, pos, rows,
               tstage, totals, shared_totals, gsem, ssem):
    sid = lax.axis_index("s")
    wid = lax.axis_index("c") * NS + sid
    chunkid = sid % W_PER_ROW
    rowhalf = sid // W_PER_ROW

    pltpu.sync_copy(mask_hbm.at[pl.ds(wid * CHUNK, CHUNK)], mchunk)

    def _acc(i, acc):
        return acc + mchunk[pl.ds(i * L, L)]

    acc = lax.fori_loop(0, NBLK, _acc, jnp.zeros((L,), jnp.int32))
    total = jnp.sum(acc)

    tstage[...] = jnp.broadcast_to(total, (L,))
    pltpu.sync_copy(tstage, shared_totals.at[sid])
    plsc.subcore_barrier()
    pltpu.sync_copy(shared_totals, totals)

    iota_v = lax.iota(jnp.int32, L)
    row_v = rowhalf * W_PER_ROW + jnp.minimum(iota_v, W_PER_ROW - 1)
    tot_v = plsc.load_gather(totals, [row_v, jnp.zeros((L,), jnp.int32)])
    base0 = jnp.sum(jnp.where(iota_v < chunkid, tot_v, 0))

    def _blocks(j, base):
        for k in range(GCH // L):
            i = j * (GCH // L) + k
            v = mchunk[pl.ds(i * L, L)]
            cs = plsc.cumsum(v) + base
            q = jnp.minimum(cs, S - 1)
            alt = S + ((lax.iota(jnp.int32, L) + i * L + wid * 32) & (K - 1))
            pos[j, pl.ds(k * L, L)] = jnp.where(v > 0, q, alt)
            base = cs[L - 1]
        return base

    def _gather(j):
        return pltpu.async_copy(
            table_hbm.at[pos.at[j]], rows.at[pl.ds(j * GCH, GCH)], gsem
        )

    def _scatter(j):
        return pltpu.async_copy(
            rows.at[pl.ds(j * GCH, GCH)],
            out_hbm.at[pl.ds(wid * CHUNK + j * GCH, GCH)],
            ssem,
        )

    half = NG // 2
    base = base0
    g = []
    for j in range(half):
        base = _blocks(j, base)
        g.append(_gather(j))
    for j in range(half, NG):
        base = _blocks(j, base)
        g.append(_gather(j))
    sc = []
    for j in range(half):
        g[j].wait()
    for j in range(half):
        sc.append(_scatter(j))
    for j in range(half, NG):
        g[j].wait()
    for j in range(half, NG):
        sc.append(_scatter(j))
    for cp in sc:
        cp.wait()


def kernel(input, mask, table):
    del input
    table_aug = jnp.concatenate(
        [table, jnp.broadcast_to(table[0:1], (K, E))], axis=0
    )
    out = _pos_embed(mask.reshape(-1), table_aug)
    return out.reshape(B, S, E)


# speedup vs baseline: 1.0570x; 1.0060x over previous
import functools

import jax
import jax.numpy as jnp
from jax import lax
from jax.experimental import pallas as pl
from jax.experimental.pallas import tpu as pltpu
from jax.experimental.pallas import tpu_sc as plsc

B, S, E = 4, 8192, 64
L = 16
NC, NS = 2, 16
NW = NC * NS
CHUNK = (B * S) // NW
W_PER_ROW = NW // B
NBLK = CHUNK // L
GCH = 128
NG = CHUNK // GCH
K = 1024

_mesh = plsc.VectorSubcoreMesh(core_axis_name="c", subcore_axis_name="s")


@functools.partial(
    pl.kernel,
    out_type=jax.ShapeDtypeStruct((B * S, E), jnp.float32),
    mesh=_mesh,
    scratch_types=[
        pltpu.VMEM((CHUNK,), jnp.int32),
        pltpu.VMEM((NG, GCH), jnp.int32),
        pltpu.VMEM((CHUNK, E), jnp.float32),
        pltpu.VMEM((L,), jnp.int32),
        pltpu.VMEM((NS, L), jnp.int32),
        pltpu.VMEM_SHARED((NS, L), jnp.int32),
        pltpu.SemaphoreType.DMA,
        pltpu.SemaphoreType.DMA,
    ],
    compiler_params=pltpu.CompilerParams(
        needs_layout_passes=False, use_tc_tiling_on_sc=False
    ),
)
def _pos_embed(mask_hbm, table_hbm, out_hbm, mchunk, pos, rows,
               tstage, totals, shared_totals, gsem, ssem):
    sid = lax.axis_index("s")
    wid = lax.axis_index("c") * NS + sid
    chunkid = sid % W_PER_ROW
    rowhalf = sid // W_PER_ROW

    pltpu.sync_copy(mask_hbm.at[pl.ds(wid * CHUNK, CHUNK)], mchunk)

    def _acc(i, acc):
        return acc + mchunk[pl.ds(i * L, L)]

    acc = lax.fori_loop(0, NBLK, _acc, jnp.zeros((L,), jnp.int32))
    total = jnp.sum(acc)

    tstage[...] = jnp.broadcast_to(total, (L,))
    pltpu.sync_copy(tstage, shared_totals.at[sid])
    plsc.subcore_barrier()
    pltpu.sync_copy(shared_totals, totals)

    iota_v = lax.iota(jnp.int32, L)
    row_v = rowhalf * W_PER_ROW + jnp.minimum(iota_v, W_PER_ROW - 1)
    tot_v = plsc.load_gather(totals, [row_v, jnp.zeros((L,), jnp.int32)])
    base0 = jnp.sum(jnp.where(iota_v < chunkid, tot_v, 0))

    def _blocks(j, base):
        for k in range(GCH // L):
            i = j * (GCH // L) + k
            v = mchunk[pl.ds(i * L, L)]
            cs = plsc.cumsum(v) + base
            q = jnp.minimum(cs, S - 1)
            alt = S + ((lax.iota(jnp.int32, L) + i * L + wid * 8) & (K - 1))
            pos[j, pl.ds(k * L, L)] = jnp.where(v > 0, q, alt)
            base = cs[L - 1]
        return base

    def _gather(j):
        return pltpu.async_copy(
            table_hbm.at[pos.at[j]], rows.at[pl.ds(j * GCH, GCH)], gsem
        )

    def _scatter(j):
        return pltpu.async_copy(
            rows.at[pl.ds(j * GCH, GCH)],
            out_hbm.at[pl.ds(wid * CHUNK + j * GCH, GCH)],
            ssem,
        )

    half = NG // 2
    base = base0
    g = []
    for j in range(half):
        base = _blocks(j, base)
        g.append(_gather(j))
    for j in range(half, NG):
        base = _blocks(j, base)
        g.append(_gather(j))
    sc = []
    for j in range(half):
        g[j].wait()
    for j in range(half):
        sc.append(_scatter(j))
    for j in range(half, NG):
        g[j].wait()
    for j in range(half, NG):
        sc.append(_scatter(j))
    for cp in sc:
        cp.wait()


def kernel(input, mask, table):
    del input
    table_aug = jnp.concatenate(
        [table, jnp.broadcast_to(table[0:1], (K, E))], axis=0
    )
    out = _pos_embed(mask.reshape(-1), table_aug)
    return out.reshape(B, S, E)


# speedup vs baseline: 1.0765x; 1.0184x over previous
import functools

import jax
import jax.numpy as jnp
from jax import lax
from jax.experimental import pallas as pl
from jax.experimental.pallas import tpu as pltpu
from jax.experimental.pallas import tpu_sc as plsc

B, S, E = 4, 8192, 64
L = 16
NC, NS = 2, 16
NW = NC * NS
CHUNK = (B * S) // NW
W_PER_ROW = NW // B
NBLK = CHUNK // L
GCH = 128
NG = CHUNK // GCH
K = 1024

_mesh = plsc.VectorSubcoreMesh(core_axis_name="c", subcore_axis_name="s")


@functools.partial(
    pl.kernel,
    out_type=jax.ShapeDtypeStruct((B * S, E), jnp.float32),
    mesh=_mesh,
    scratch_types=[
        pltpu.VMEM((CHUNK,), jnp.int32),
        pltpu.VMEM((NG, GCH), jnp.int32),
        pltpu.VMEM((CHUNK, E), jnp.float32),
        pltpu.VMEM((L,), jnp.int32),
        pltpu.VMEM((NS, L), jnp.int32),
        pltpu.VMEM_SHARED((NS, L), jnp.int32),
        pltpu.SemaphoreType.DMA,
        pltpu.SemaphoreType.DMA,
        pltpu.SemaphoreType.DMA,
    ],
    compiler_params=pltpu.CompilerParams(
        needs_layout_passes=False, use_tc_tiling_on_sc=False
    ),
)
def _pos_embed(mask_hbm, table_hbm, out_hbm, mchunk, pos, rows,
               tstage, totals, shared_totals, gsemA, gsemB, ssem):
    sid = lax.axis_index("s")
    wid = lax.axis_index("c") * NS + sid
    chunkid = sid % W_PER_ROW
    rowhalf = sid // W_PER_ROW

    pltpu.sync_copy(mask_hbm.at[pl.ds(wid * CHUNK, CHUNK)], mchunk)

    def _acc(i, acc):
        return acc + mchunk[pl.ds(i * L, L)]

    acc = lax.fori_loop(0, NBLK, _acc, jnp.zeros((L,), jnp.int32))
    total = jnp.sum(acc)

    tstage[...] = jnp.broadcast_to(total, (L,))
    pltpu.sync_copy(tstage, shared_totals.at[sid])
    plsc.subcore_barrier()
    pltpu.sync_copy(shared_totals, totals)

    iota_v = lax.iota(jnp.int32, L)
    row_v = rowhalf * W_PER_ROW + jnp.minimum(iota_v, W_PER_ROW - 1)
    tot_v = plsc.load_gather(totals, [row_v, jnp.zeros((L,), jnp.int32)])
    base0 = jnp.sum(jnp.where(iota_v < chunkid, tot_v, 0))

    def _blocks(j, base):
        for k in range(GCH // L):
            i = j * (GCH // L) + k
            v = mchunk[pl.ds(i * L, L)]
            cs = plsc.cumsum(v) + base
            q = jnp.minimum(cs, S - 1)
            alt = S + ((lax.iota(jnp.int32, L) + i * L + wid * 8) & (K - 1))
            pos[j, pl.ds(k * L, L)] = jnp.where(v > 0, q, alt)
            base = cs[L - 1]
        return base

    def _gather(j):
        sem = gsemA if j < NG // 2 else gsemB
        return pltpu.async_copy(
            table_hbm.at[pos.at[j]], rows.at[pl.ds(j * GCH, GCH)], sem
        )

    def _scatter(j):
        return pltpu.async_copy(
            rows.at[pl.ds(j * GCH, GCH)],
            out_hbm.at[pl.ds(wid * CHUNK + j * GCH, GCH)],
            ssem,
        )

    half = NG // 2
    base = base0
    g = []
    for j in range(half):
        base = _blocks(j, base)
        g.append(_gather(j))
    for j in range(half, NG):
        base = _blocks(j, base)
        g.append(_gather(j))
    sc = []
    for j in range(half):
        g[j].wait()
    for j in range(half):
        sc.append(_scatter(j))
    for j in range(half, NG):
        g[j].wait()
    for j in range(half, NG):
        sc.append(_scatter(j))
    for cp in sc:
        cp.wait()


def kernel(input, mask, table):
    del input
    table_aug = jnp.concatenate(
        [table, jnp.broadcast_to(table[0:1], (K, E))], axis=0
    )
    out = _pos_embed(mask.reshape(-1), table_aug)
    return out.reshape(B, S, E)
